# baseline (speedup 1.0000x reference)
import functools

import jax
import jax.numpy as jnp
from jax import lax
from jax.experimental import pallas as pl
from jax.experimental.pallas import tpu as pltpu
from jax.experimental.pallas import tpu_sc as plsc

B = 16384
F = 26
V = 100000
D = 16
N_DENSE = 13
SP = F * D

R = B * F
NW = 32
G = 128
K = 8
CH = G * K
R_PER_W = R // NW
NG = R_PER_W // G
NCH = R_PER_W // CH


def _gather_body(idx_hbm, table_hbm, out_hbm, idx_v, rows_v, gsem):
    wid = lax.axis_index("s") * 2 + lax.axis_index("c")
    base_g = wid * NG
    base_row = wid * R_PER_W
    pltpu.sync_copy(idx_hbm.at[pl.ds(base_g, NG)], idx_v)

    def chunk(t, carry):
        descs = []
        for j in range(K):
            descs.append(
                pltpu.async_copy(
                    table_hbm.at[idx_v.at[t * K + j]],
                    rows_v.at[pl.ds(j * G, G)],
                    gsem,
                )
            )
        for dsc in descs:
            dsc.wait()
        pltpu.sync_copy(rows_v, out_hbm.at[pl.ds(base_row + t * CH, CH)])
        return carry

    lax.fori_loop(0, NCH, chunk, 0)


@functools.cache
def _gather():
    return pl.kernel(
        _gather_body,
        out_type=jax.ShapeDtypeStruct((R, D), jnp.float32),
        mesh=plsc.VectorSubcoreMesh(core_axis_name="c", subcore_axis_name="s"),
        scratch_types=[
            pltpu.VMEM((NG, G), jnp.int32),
            pltpu.VMEM((CH, D), jnp.float32),
            pltpu.SemaphoreType.DMA,
        ],
        compiler_params=pltpu.CompilerParams(use_tc_tiling_on_sc=False),
    )


BB = 1024


def _tc_body(xs_ref, xd_ref, w1s_ref, w1d_ref, b1_ref, w2_ref, b2_ref,
             w3_ref, b3_ref, wd_ref, bd_ref, wfs_ref, wfd_ref, bfm_ref,
             o_ref):
    f32 = jnp.float32
    dot = lambda a, b: lax.dot_general(
        a, b, (((1,), (0,)), ((), ())),
        preferred_element_type=f32, precision=lax.Precision.HIGHEST)
    xs = xs_ref[...]
    xd = xd_ref[...]
    ci = lax.broadcasted_iota(jnp.int32, (SP, D), 0)
    cj = lax.broadcasted_iota(jnp.int32, (SP, D), 1)
    sel = jnp.where((ci % D) == cj, 1.0, 0.0).astype(f32)
    s1 = dot(xs, sel)
    s2 = dot(xs * xs, sel)
    fm2 = 0.5 * jnp.sum(s1 * s1 - s2, axis=1, keepdims=True)
    fm1 = dot(xs, wfs_ref[...]) + dot(xd, wfd_ref[...]) + bfm_ref[...]
    h = jnp.maximum(dot(xs, w1s_ref[...]) + dot(xd, w1d_ref[...]) + b1_ref[...], 0.0)
    h = jnp.maximum(dot(h, w2_ref[...]) + b2_ref[...], 0.0)
    h = jnp.maximum(dot(h, w3_ref[...]) + b3_ref[...], 0.0)
    dnn = dot(h, wd_ref[...]) + bd_ref[...]
    o_ref[...] = jax.nn.sigmoid(fm1 + fm2 + dnn)


def _full(shape):
    return pl.BlockSpec(shape, lambda i: (0, 0))


_tc_call = pl.pallas_call(
    _tc_body,
    grid=(B // BB,),
    in_specs=[
        pl.BlockSpec((BB, SP), lambda i: (i, 0)),
        pl.BlockSpec((BB, N_DENSE), lambda i: (i, 0)),
        _full((SP, 256)),
        _full((N_DENSE, 256)),
        _full((1, 256)),
        _full((256, 128)),
        _full((1, 128)),
        _full((128, 64)),
        _full((1, 64)),
        _full((64, 1)),
        _full((1, 1)),
        _full((SP, 1)),
        _full((N_DENSE, 1)),
        _full((1, 1)),
    ],
    out_specs=pl.BlockSpec((BB, 1), lambda i: (i, 0)),
    out_shape=jax.ShapeDtypeStruct((B, 1), jnp.float32),
)


def kernel(dense_input, sparse_input, embed_tables, W_fm, b_fm,
           W1, b1, W2, b2, W3, b3, Wd, bd):
    table_flat = embed_tables.reshape(F * V, D)
    flat_idx = (sparse_input.astype(jnp.int32)
                + (jnp.arange(F, dtype=jnp.int32) * V)[None, :]).reshape(R // G, G)
    rows = _gather()(flat_idx, table_flat)
    xs = rows.reshape(B, SP)
    out = _tc_call(
        xs, dense_input,
        W1[:SP], W1[SP:], b1.reshape(1, -1),
        W2, b2.reshape(1, -1),
        W3, b3.reshape(1, -1),
        Wd, bd.reshape(1, -1),
        W_fm[:SP], W_fm[SP:], b_fm.reshape(1, -1),
    )
    return out.reshape(B)


# speedup vs baseline: 4.9377x; 4.9377x over previous
import functools

import jax
import jax.numpy as jnp
from jax import lax
from jax.experimental import pallas as pl
from jax.experimental.pallas import tpu as pltpu
from jax.experimental.pallas import tpu_sc as plsc

B = 16384
F = 26
V = 100000
D = 16
N_DENSE = 13
SP = F * D

NW = 32
ROWS_PER_W = SP // NW
HALF = B // 2
UNROLL = 8


def _gather_body(idxT_hbm, table_hbm, out_hbm, idx_v, row_v, out_v):
    wid = lax.axis_index("s") * 2 + lax.axis_index("c")

    def do_row(j, carry):
        r = wid * ROWS_PER_W + j
        f = r // D
        pltpu.sync_copy(table_hbm.at[r], row_v)

        def do_half(h, carry2):
            pltpu.sync_copy(idxT_hbm.at[f, pl.ds(h * HALF, HALF)], idx_v)

            def gblk(i, carry3):
                for u in range(UNROLL):
                    off = (i * UNROLL + u) * 16
                    iv = idx_v[pl.ds(off, 16)]
                    out_v[pl.ds(h * HALF + off, 16)] = plsc.load_gather(row_v, [iv])
                return carry3

            lax.fori_loop(0, HALF // (16 * UNROLL), gblk, 0)
            return carry2

        lax.fori_loop(0, 2, do_half, 0)
        pltpu.sync_copy(out_v, out_hbm.at[r])
        return carry

    lax.fori_loop(0, ROWS_PER_W, do_row, 0)


@functools.cache
def _gather():
    return pl.kernel(
        _gather_body,
        out_type=jax.ShapeDtypeStruct((SP, B), jnp.float32),
        mesh=plsc.VectorSubcoreMesh(core_axis_name="c", subcore_axis_name="s"),
        scratch_types=[
            pltpu.VMEM((HALF,), jnp.int32),
            pltpu.VMEM((V,), jnp.float32),
            pltpu.VMEM((B,), jnp.float32),
        ],
        compiler_params=pltpu.CompilerParams(needs_layout_passes=False),
    )


BB = 2048


def _tc_body(xs_ref, xd_ref, w1_ref, b1_ref, w2_ref, b2_ref,
             w3t_ref, b3_ref, wdt_ref, bd_ref, wfmt_ref, bfm_ref,
             o_ref):
    f32 = jnp.float32
    prec = lax.Precision.HIGHEST

    def dott(a, b):
        return lax.dot_general(a, b, (((0,), (0,)), ((), ())),
                               preferred_element_type=f32, precision=prec)

    def dotn(a, b):
        return lax.dot_general(a, b, (((1,), (0,)), ((), ())),
                               preferred_element_type=f32, precision=prec)

    xs = xs_ref[...]
    xd = xd_ref[...]
    ci = lax.broadcasted_iota(jnp.int32, (D, SP), 0)
    cj = lax.broadcasted_iota(jnp.int32, (D, SP), 1)
    sel = jnp.where((cj % D) == ci, 1.0, 0.0).astype(f32)
    s1 = dotn(sel, xs)
    s2 = dotn(sel, xs * xs)
    fm2 = 0.5 * jnp.sum(s1 * s1 - s2, axis=0, keepdims=True)
    wfmt = wfmt_ref[...]
    fm1 = dotn(wfmt[:, :SP], xs) + dotn(wfmt[:, SP:], xd) + bfm_ref[...]
    w1 = w1_ref[...]
    h = jnp.maximum(dott(w1[:SP], xs) + dott(w1[SP:], xd) + b1_ref[...], 0.0)
    h = jnp.maximum(dott(w2_ref[...], h) + b2_ref[...], 0.0)
    h = jnp.maximum(dotn(w3t_ref[...], h) + b3_ref[...], 0.0)
    dnn = dotn(wdt_ref[...], h) + bd_ref[...]
    o_ref[...] = jax.nn.sigmoid(fm1 + fm2 + dnn)


def _full(shape):
    return pl.BlockSpec(shape, lambda i: tuple(0 for _ in shape))


_tc_call = pl.pallas_call(
    _tc_body,
    grid=(B // BB,),
    in_specs=[
        pl.BlockSpec((SP, BB), lambda i: (0, i)),
        pl.BlockSpec((N_DENSE, BB), lambda i: (0, i)),
        _full((SP + N_DENSE, 256)),
        _full((256, 1)),
        _full((256, 128)),
        _full((128, 1)),
        _full((64, 128)),
        _full((64, 1)),
        _full((1, 64)),
        _full((1, 1)),
        _full((1, SP + N_DENSE)),
        _full((1, 1)),
    ],
    out_specs=pl.BlockSpec((1, BB), lambda i: (0, i)),
    out_shape=jax.ShapeDtypeStruct((1, B), jnp.float32),
)


def kernel(dense_input, sparse_input, embed_tables, W_fm, b_fm,
           W1, b1, W2, b2, W3, b3, Wd, bd):
    tableT = embed_tables.transpose(0, 2, 1).reshape(SP, V)
    idxT = sparse_input.T
    xsT = _gather()(idxT, tableT)
    outT = _tc_call(
        xsT, dense_input.T,
        W1, b1.reshape(-1, 1),
        W2, b2.reshape(-1, 1),
        W3.T, b3.reshape(-1, 1),
        Wd.T, bd.reshape(1, 1),
        W_fm.T, b_fm.reshape(1, 1),
    )
    return outT.reshape(B)


# speedup vs baseline: 6.7946x; 1.3760x over previous
import functools

import jax
import jax.numpy as jnp
from jax import lax
from jax.experimental import pallas as pl
from jax.experimental.pallas import tpu as pltpu
from jax.experimental.pallas import tpu_sc as plsc

B = 16384
F = 26
V = 100000
D = 16
N_DENSE = 13
SP = F * D

NW = 32
ROWS_PER_W = SP // NW
HALF = B // 2
UNROLL = 8


def _gather_body(idxT_hbm, table_hbm, out_hbm, idx_v, row_v, out_v):
    wid = lax.axis_index("s") * 2 + lax.axis_index("c")

    def do_row(j, carry):
        r = wid * ROWS_PER_W + j
        f = r // D
        pltpu.sync_copy(table_hbm.at[r], row_v)

        def do_half(h, carry2):
            pltpu.sync_copy(idxT_hbm.at[f, pl.ds(h * HALF, HALF)], idx_v)

            def gblk(i, carry3):
                for u in range(UNROLL):
                    off = (i * UNROLL + u) * 16
                    iv = idx_v[pl.ds(off, 16)]
                    out_v[pl.ds(h * HALF + off, 16)] = plsc.load_gather(row_v, [iv])
                return carry3

            lax.fori_loop(0, HALF // (16 * UNROLL), gblk, 0)
            return carry2

        lax.fori_loop(0, 2, do_half, 0)
        pltpu.sync_copy(out_v, out_hbm.at[r])
        return carry

    lax.fori_loop(0, ROWS_PER_W, do_row, 0)


@functools.cache
def _gather():
    return pl.kernel(
        _gather_body,
        out_type=jax.ShapeDtypeStruct((SP, B), jnp.float32),
        mesh=plsc.VectorSubcoreMesh(core_axis_name="c", subcore_axis_name="s"),
        scratch_types=[
            pltpu.VMEM((HALF,), jnp.int32),
            pltpu.VMEM((V,), jnp.float32),
            pltpu.VMEM((B,), jnp.float32),
        ],
        compiler_params=pltpu.CompilerParams(needs_layout_passes=False),
    )


BB = 2048


def _tc_body(xs_ref, xd_ref, w1_ref, b1_ref, w2_ref, b2_ref,
             w3t_ref, b3_ref, wdt_ref, bd_ref, wfmt_ref, bfm_ref,
             o_ref):
    f32 = jnp.float32
    prec = lax.Precision.DEFAULT

    def dott(a, b):
        return lax.dot_general(a, b, (((0,), (0,)), ((), ())),
                               preferred_element_type=f32, precision=prec)

    def dotn(a, b):
        return lax.dot_general(a, b, (((1,), (0,)), ((), ())),
                               preferred_element_type=f32, precision=prec)

    xs = xs_ref[...]
    xd = xd_ref[...]
    ci = lax.broadcasted_iota(jnp.int32, (D, SP), 0)
    cj = lax.broadcasted_iota(jnp.int32, (D, SP), 1)
    sel = jnp.where((cj % D) == ci, 1.0, 0.0).astype(f32)
    s1 = dotn(sel, xs)
    s2 = dotn(sel, xs * xs)
    fm2 = 0.5 * jnp.sum(s1 * s1 - s2, axis=0, keepdims=True)
    wfmt = wfmt_ref[...]
    fm1 = dotn(wfmt[:, :SP], xs) + dotn(wfmt[:, SP:], xd) + bfm_ref[...]
    w1 = w1_ref[...]
    h = jnp.maximum(dott(w1[:SP], xs) + dott(w1[SP:], xd) + b1_ref[...], 0.0)
    h = jnp.maximum(dott(w2_ref[...], h) + b2_ref[...], 0.0)
    h = jnp.maximum(dotn(w3t_ref[...], h) + b3_ref[...], 0.0)
    dnn = dotn(wdt_ref[...], h) + bd_ref[...]
    o_ref[...] = jax.nn.sigmoid(fm1 + fm2 + dnn)


def _full(shape):
    return pl.BlockSpec(shape, lambda i: tuple(0 for _ in shape))


_tc_call = pl.pallas_call(
    _tc_body,
    grid=(B // BB,),
    in_specs=[
        pl.BlockSpec((SP, BB), lambda i: (0, i)),
        pl.BlockSpec((N_DENSE, BB), lambda i: (0, i)),
        _full((SP + N_DENSE, 256)),
        _full((256, 1)),
        _full((256, 128)),
        _full((128, 1)),
        _full((64, 128)),
        _full((64, 1)),
        _full((1, 64)),
        _full((1, 1)),
        _full((1, SP + N_DENSE)),
        _full((1, 1)),
    ],
    out_specs=pl.BlockSpec((1, BB), lambda i: (0, i)),
    out_shape=jax.ShapeDtypeStruct((1, B), jnp.float32),
)


def kernel(dense_input, sparse_input, embed_tables, W_fm, b_fm,
           W1, b1, W2, b2, W3, b3, Wd, bd):
    tableT = embed_tables.transpose(0, 2, 1).reshape(SP, V)
    idxT = sparse_input.T
    xsT = _gather()(idxT, tableT)
    outT = _tc_call(
        xsT, dense_input.T,
        W1, b1.reshape(-1, 1),
        W2, b2.reshape(-1, 1),
        W3.T, b3.reshape(-1, 1),
        Wd.T, bd.reshape(1, 1),
        W_fm.T, b_fm.reshape(1, 1),
    )
    return outT.reshape(B)
